# baseline (speedup 1.0000x reference)
import functools

import jax
import jax.numpy as jnp
from jax import lax
from jax.experimental import pallas as pl
from jax.experimental.pallas import tpu as pltpu
from jax.experimental.pallas import tpu_sc as plsc

VOCAB = 100000
DIM = 64
BATCH = 16384
NUM_SAMPLES = 2

import numpy as np

NC = 2
NS = 16
NW = NC * NS
BPW = BATCH // NW
GROUPS = BPW // 16

_BFLY = tuple((np.arange(16) ^ (1 << k)).astype(np.int32) for k in range(4))
_LANE = np.arange(16, dtype=np.int32)


def _sc_neg_scores(table_hbm, pred_hbm, q_hbm, j_hbm,
                   r0_hbm, u0_hbm, r1_hbm, u1_hbm, out_hbm,
                   pred_v, r_v, u_v, q_v, j_v, idx_v, rows_v, scores_v, sem):
    wid = lax.axis_index("s") * NC + lax.axis_index("c")
    base = pl.multiple_of(wid * BPW, BPW)
    pltpu.sync_copy(pred_hbm.at[pl.ds(base, BPW)], pred_v)
    for i, (r_hbm, u_hbm) in enumerate(((r0_hbm, u0_hbm), (r1_hbm, u1_hbm))):
        pltpu.sync_copy(r_hbm.at[pl.ds(base, BPW)], r_v)
        pltpu.sync_copy(u_hbm.at[pl.ds(base, BPW)], u_v)
        pltpu.async_copy(q_hbm.at[r_v], q_v, sem).wait()
        pltpu.async_copy(j_hbm.at[r_v], j_v, sem).wait()

        def sel_body(c, _):
            sl = pl.ds(c * 16, 16)
            qq = jnp.minimum(jnp.maximum(q_v[sl], 0.0), 1.0)
            keep = u_v[sl] < qq
            idx_v[sl] = jnp.where(keep, r_v[sl], j_v[sl])
            return 0

        lax.fori_loop(0, GROUPS, sel_body, 0)

        pltpu.async_copy(table_hbm.at[idx_v], rows_v, sem).wait()

        def dot_body(g, _):
            lane = lax.iota(jnp.int32, 16)
            acc = jnp.zeros((16,), jnp.float32)
            for l in range(16):
                j = g * 16 + l
                pacc = jnp.zeros((16,), jnp.float32)
                for c in range(DIM // 16):
                    sl = pl.ds(c * 16, 16)
                    pacc = pacc + pred_v[j, sl] * rows_v[j, sl]
                for k in range(4):
                    pacc = pacc + jnp.take_along_axis(
                        pacc, lane ^ (1 << k), axis=0,
                        mode="promise_in_bounds")
                acc = acc + jnp.where(lane == l, pacc, 0.0)
            scores_v[pl.ds(g * 16, 16)] = acc
            return 0

        lax.fori_loop(0, GROUPS, dot_body, 0)
        pltpu.sync_copy(scores_v, out_hbm.at[i, pl.ds(base, BPW)])


@functools.lru_cache(maxsize=None)
def _sc_call():
    return functools.partial(
        pl.kernel,
        mesh=plsc.VectorSubcoreMesh(core_axis_name="c", subcore_axis_name="s"),
        compiler_params=pltpu.CompilerParams(use_tc_tiling_on_sc=False),
        out_type=jax.ShapeDtypeStruct((NUM_SAMPLES, BATCH), jnp.float32),
        scratch_types=[
            pltpu.VMEM((BPW, DIM), jnp.float32),
            pltpu.VMEM((BPW,), jnp.int32),
            pltpu.VMEM((BPW,), jnp.float32),
            pltpu.VMEM((BPW,), jnp.float32),
            pltpu.VMEM((BPW,), jnp.int32),
            pltpu.VMEM((BPW,), jnp.int32),
            pltpu.VMEM((BPW, DIM), jnp.float32),
            pltpu.VMEM((BPW,), jnp.float32),
            pltpu.SemaphoreType.DMA,
        ],
    )(_sc_neg_scores)


def _tc_loss(pred_ref, tgt_ref, neg_ref, out_ref):
    p = pred_ref[...]
    t = tgt_ref[...]
    ts = jnp.sum(p * t, axis=1)
    l1 = jnp.maximum(ts, 0.0) - ts + jnp.log1p(jnp.exp(-jnp.abs(ts)))
    n = neg_ref[...]
    ln = jnp.maximum(n, 0.0) + jnp.log1p(jnp.exp(-jnp.abs(n)))
    total = (jnp.sum(l1) + jnp.sum(ln)) / jnp.float32(BATCH)
    out_ref[...] = jnp.reshape(total, (1, 1))


def _draws():
    rs, us = [], []
    for i in range(NUM_SAMPLES):
        key = jax.random.fold_in(jax.random.key(1), i)
        kr, kb = jax.random.split(key)
        rs.append(jax.random.randint(kr, (BATCH,), 0, VOCAB, dtype=jnp.int32))
        us.append(jax.random.uniform(kb, (BATCH,)))
    return rs, us


def kernel(predicted, target, table, alias_q, alias_J):
    predicted = jnp.squeeze(predicted)
    target = jnp.squeeze(target)
    (r0, r1), (u0, u1) = _draws()
    neg = _sc_call()(table, predicted, alias_q, alias_J.astype(jnp.int32),
                     r0, u0, r1, u1)
    loss = pl.pallas_call(
        _tc_loss,
        out_shape=jax.ShapeDtypeStruct((1, 1), jnp.float32),
    )(predicted, target, neg)
    return loss[0, 0]
